# baseline (speedup 1.0000x reference)
import functools

import jax
import jax.numpy as jnp
from jax import lax
from jax.experimental import pallas as pl
from jax.experimental.pallas import tpu as pltpu
from jax.experimental.pallas import tpu_sc as plsc

_INPUT_SIZE = 100000
_D = 64
_HIDDEN = 128
_OUT = 128
_BATCH = 16384

_NC, _NS = 2, 16
_NW = _NC * _NS
_B2 = 2 * _BATCH
_BPW = _B2 // _NW

_BB = 2048
_NBLK = _BATCH // _BB


def _sc_gather(emb, idx):
    mesh = plsc.VectorSubcoreMesh(core_axis_name="c", subcore_axis_name="s")

    @functools.partial(
        pl.kernel,
        out_type=jax.ShapeDtypeStruct((_B2, _D), jnp.float32),
        mesh=mesh,
        scratch_types=[
            pltpu.VMEM((_BPW,), jnp.int32),
            pltpu.VMEM((_BPW, _D), jnp.float32),
            pltpu.SemaphoreType.DMA,
        ],
        compiler_params=pltpu.CompilerParams(use_tc_tiling_on_sc=False),
    )
    def k(emb_hbm, idx_hbm, out_hbm, idx_v, rows_v, sem):
        wid = lax.axis_index("s") * _NC + lax.axis_index("c")
        base = wid * _BPW
        pltpu.sync_copy(idx_hbm.at[pl.ds(base, _BPW)], idx_v)
        pltpu.async_copy(emb_hbm.at[idx_v], rows_v, sem).wait()
        pltpu.sync_copy(rows_v, out_hbm.at[pl.ds(base, _BPW)])

    return k(emb, idx)


def _tc_mlp(g, W1, b1, W2, b2):
    def body(g0_ref, g1_ref, w1_ref, b1_ref, w2_ref, b2_ref, o_ref):
        h = g0_ref[...] + g1_ref[...]
        h = jnp.dot(h, w1_ref[...], preferred_element_type=jnp.float32)
        h = h + b1_ref[...]
        h = h * h
        o = jnp.dot(h, w2_ref[...], preferred_element_type=jnp.float32)
        o_ref[...] = o + b2_ref[...]

    return pl.pallas_call(
        body,
        grid=(_NBLK,),
        in_specs=[
            pl.BlockSpec((_BB, _D), lambda i: (i, 0)),
            pl.BlockSpec((_BB, _D), lambda i: (i + _NBLK, 0)),
            pl.BlockSpec((_D, _HIDDEN), lambda i: (0, 0)),
            pl.BlockSpec((1, _HIDDEN), lambda i: (0, 0)),
            pl.BlockSpec((_HIDDEN, _OUT), lambda i: (0, 0)),
            pl.BlockSpec((1, _OUT), lambda i: (0, 0)),
        ],
        out_specs=pl.BlockSpec((_BB, _OUT), lambda i: (i, 0)),
        out_shape=jax.ShapeDtypeStruct((_BATCH, _OUT), jnp.float32),
    )(g, g, W1, b1, W2, b2)


def kernel(x, emb, W1, b1, W2, b2):
    idx = jnp.concatenate([x[:, 0], x[:, 1] + _INPUT_SIZE])
    g = _sc_gather(emb, idx)
    out0 = _tc_mlp(g, W1, b1.reshape(1, _HIDDEN), W2, b2.reshape(1, _OUT))
    return (out0,)


# speedup vs baseline: 1.7572x; 1.7572x over previous
import functools

import jax
import jax.numpy as jnp
from jax import lax
from jax.experimental import pallas as pl
from jax.experimental.pallas import tpu as pltpu
from jax.experimental.pallas import tpu_sc as plsc

_INPUT_SIZE = 100000
_N = 2 * _INPUT_SIZE
_D = 64
_HIDDEN = 128
_OUT = 128
_BATCH = 16384

_NC, _NS = 2, 16
_NW = _NC * _NS
_B2 = 2 * _BATCH
_BPW = _B2 // _NW
_GCHUNK = 512
_NCHUNK = _BPW // _GCHUNK

_BN = 8192
_BB = 2048


def _tc_table_matmul(embT, W1, b1_half):
    def body(embT_ref, w1_ref, b1_ref, y_ref):
        y = lax.dot_general(
            embT_ref[...], w1_ref[...],
            dimension_numbers=(((0,), (0,)), ((), ())),
            preferred_element_type=jnp.float32,
        )
        y_ref[...] = y + b1_ref[...]

    return pl.pallas_call(
        body,
        grid=(pl.cdiv(_N, _BN),),
        in_specs=[
            pl.BlockSpec((_D, _BN), lambda i: (0, i)),
            pl.BlockSpec((_D, _HIDDEN), lambda i: (0, 0)),
            pl.BlockSpec((1, _HIDDEN), lambda i: (0, 0)),
        ],
        out_specs=pl.BlockSpec((_BN, _HIDDEN), lambda i: (i, 0)),
        out_shape=jax.ShapeDtypeStruct((_N, _HIDDEN), jnp.float32),
    )(embT, W1, b1_half)


def _sc_gather(y, idx):
    mesh = plsc.VectorSubcoreMesh(core_axis_name="c", subcore_axis_name="s")

    @functools.partial(
        pl.kernel,
        out_type=jax.ShapeDtypeStruct((_B2, _HIDDEN), jnp.float32),
        mesh=mesh,
        scratch_types=[
            pltpu.VMEM((_GCHUNK,), jnp.int32),
            pltpu.VMEM((_GCHUNK,), jnp.int32),
            pltpu.VMEM((_GCHUNK, _HIDDEN), jnp.float32),
            pltpu.SemaphoreType.DMA,
        ],
    )
    def k(y_hbm, idx_hbm, out_hbm, idx_v0, idx_v1, rows_v, sem):
        wid = lax.axis_index("s") * _NC + lax.axis_index("c")
        base = wid * _BPW
        for c, idx_v in enumerate((idx_v0, idx_v1)):
            pltpu.sync_copy(
                idx_hbm.at[pl.ds(base + c * _GCHUNK, _GCHUNK)], idx_v)
            pltpu.async_copy(y_hbm.at[idx_v], rows_v, sem).wait()
            pltpu.sync_copy(rows_v, out_hbm.at[pl.ds(base + c * _GCHUNK, _GCHUNK)])

    return k(y, idx)


def _tc_pair_mlp(g, W2, b2):
    def body(g0_ref, g1_ref, w2_ref, b2_ref, o_ref):
        h = g0_ref[...] + g1_ref[...]
        h = h * h
        o = jnp.dot(h, w2_ref[...], preferred_element_type=jnp.float32)
        o_ref[...] = o + b2_ref[...]

    nblk = _BATCH // _BB
    return pl.pallas_call(
        body,
        grid=(nblk,),
        in_specs=[
            pl.BlockSpec((_BB, _HIDDEN), lambda i: (i, 0)),
            pl.BlockSpec((_BB, _HIDDEN), lambda i: (i + nblk, 0)),
            pl.BlockSpec((_HIDDEN, _OUT), lambda i: (0, 0)),
            pl.BlockSpec((1, _OUT), lambda i: (0, 0)),
        ],
        out_specs=pl.BlockSpec((_BB, _OUT), lambda i: (i, 0)),
        out_shape=jax.ShapeDtypeStruct((_BATCH, _OUT), jnp.float32),
    )(g, g, W2, b2)


def kernel(x, emb, W1, b1, W2, b2):
    idx = jnp.concatenate([x[:, 0], x[:, 1] + _INPUT_SIZE])
    y = _tc_table_matmul(emb.T, W1, (0.5 * b1).reshape(1, _HIDDEN))
    g = _sc_gather(y, idx)
    out0 = _tc_pair_mlp(g, W2, b2.reshape(1, _OUT))
    return (out0,)


# speedup vs baseline: 1.8000x; 1.0244x over previous
import functools

import jax
import jax.numpy as jnp
from jax import lax
from jax.experimental import pallas as pl
from jax.experimental.pallas import tpu as pltpu
from jax.experimental.pallas import tpu_sc as plsc

_INPUT_SIZE = 100000
_N = 2 * _INPUT_SIZE
_D = 64
_HIDDEN = 128
_OUT = 128
_BATCH = 16384

_NC, _NS = 2, 16
_NW = _NC * _NS
_B2 = 2 * _BATCH
_BPW = _B2 // _NW
_GCHUNK = 256
_NCHUNK = _BPW // _GCHUNK

_BN = 8192
_BB = 2048


def _tc_table_matmul(embT, W1, b1):
    def body(embT_ref, w1_ref, b1_ref, y_ref):
        y = lax.dot_general(
            embT_ref[...].astype(jnp.bfloat16),
            w1_ref[...].astype(jnp.bfloat16),
            dimension_numbers=(((0,), (0,)), ((), ())),
            preferred_element_type=jnp.float32,
        )
        y_ref[...] = y + 0.5 * b1_ref[...]

    return pl.pallas_call(
        body,
        grid=(pl.cdiv(_N, _BN),),
        in_specs=[
            pl.BlockSpec((_D, _BN), lambda i: (0, i)),
            pl.BlockSpec((_D, _HIDDEN), lambda i: (0, 0)),
            pl.BlockSpec((1, _HIDDEN), lambda i: (0, 0)),
        ],
        out_specs=pl.BlockSpec((_BN, _HIDDEN), lambda i: (i, 0)),
        out_shape=jax.ShapeDtypeStruct((_N, _HIDDEN), jnp.float32),
    )(embT, W1, b1)


def _sc_gather(y, idx):
    mesh = plsc.VectorSubcoreMesh(core_axis_name="c", subcore_axis_name="s")

    @functools.partial(
        pl.kernel,
        out_type=jax.ShapeDtypeStruct((_B2, _HIDDEN), jnp.float32),
        mesh=mesh,
        scratch_types=[
            pltpu.VMEM((_GCHUNK,), jnp.int32),
            pltpu.VMEM((_GCHUNK,), jnp.int32),
            pltpu.VMEM((_GCHUNK, _HIDDEN), jnp.float32),
            pltpu.VMEM((_GCHUNK, _HIDDEN), jnp.float32),
            pltpu.SemaphoreType.DMA,
            pltpu.SemaphoreType.DMA,
            pltpu.SemaphoreType.DMA,
            pltpu.SemaphoreType.DMA,
        ],
    )
    def k(y_hbm, idx_hbm, out_hbm, idx_v0, idx_v1, rows_v0, rows_v1,
          g_sem0, g_sem1, w_sem0, w_sem1):
        wid = lax.axis_index("s") * _NC + lax.axis_index("c")
        base = wid * _BPW
        idx_bufs = (idx_v0, idx_v1)
        row_bufs = (rows_v0, rows_v1)
        g_sems = (g_sem0, g_sem1)
        w_sems = (w_sem0, w_sem1)
        writes = [None, None]
        for c in range(_NCHUNK):
            b = c % 2
            if writes[b] is not None:
                writes[b].wait()
            lo = base + c * _GCHUNK
            pltpu.sync_copy(idx_hbm.at[pl.ds(lo, _GCHUNK)], idx_bufs[b])
            pltpu.async_copy(y_hbm.at[idx_bufs[b]], row_bufs[b],
                             g_sems[b]).wait()
            writes[b] = pltpu.async_copy(
                row_bufs[b], out_hbm.at[pl.ds(lo, _GCHUNK)], w_sems[b])
        writes[0].wait()
        writes[1].wait()

    return k(y, idx)


def _tc_pair_mlp(g, W2, b2):
    def body(g0_ref, g1_ref, w2_ref, b2_ref, o_ref):
        h = g0_ref[...] + g1_ref[...]
        h = h * h
        o = jnp.dot(h.astype(jnp.bfloat16), w2_ref[...].astype(jnp.bfloat16),
                    preferred_element_type=jnp.float32)
        o_ref[...] = o + b2_ref[...]

    nblk = _BATCH // _BB
    return pl.pallas_call(
        body,
        grid=(nblk,),
        in_specs=[
            pl.BlockSpec((_BB, _HIDDEN), lambda i: (i, 0)),
            pl.BlockSpec((_BB, _HIDDEN), lambda i: (i + nblk, 0)),
            pl.BlockSpec((_HIDDEN, _OUT), lambda i: (0, 0)),
            pl.BlockSpec((1, _OUT), lambda i: (0, 0)),
        ],
        out_specs=pl.BlockSpec((_BB, _OUT), lambda i: (i, 0)),
        out_shape=jax.ShapeDtypeStruct((_BATCH, _OUT), jnp.float32),
    )(g, g, W2, b2)


def kernel(x, emb, W1, b1, W2, b2):
    idx = jnp.concatenate([x[:, 0], x[:, 1] + _INPUT_SIZE])
    y = _tc_table_matmul(emb.T, W1, b1.reshape(1, _HIDDEN))
    g = _sc_gather(y, idx)
    out0 = _tc_pair_mlp(g, W2, b2.reshape(1, _OUT))
    return (out0,)
